# Initial kernel scaffold; baseline (speedup 1.0000x reference)
import jax
import jax.numpy as jnp
from jax.experimental import pallas as pl


def kernel(emb, w_dis1, b_dis1, w_dis2, b_dis2, w_pmi1, b_pmi1, w_pmi2, b_pmi2, w_top1, b_top1, w_top2, b_top2, wq, wk, wv, wo, wd, bd, wf, bf, ei_dis, ei_pmi, ei_top, adj):
    raise NotImplementedError("write your pallas kernel here")


# speedup vs baseline: 1.8615x; 1.8615x over previous
import functools
import numpy as np

import jax
import jax.numpy as jnp
from jax import lax
from jax.experimental import pallas as pl
from jax.experimental.pallas import tpu as pltpu
from jax.experimental.pallas import tpu_sc as plsc

_N, _E, _D, _C, _B, _L, _H = 10000, 160000, 300, 20, 4096, 50, 30
_DP = 304
_BLK = 1000
_NB = _N // _BLK
_MBLK = 1256
_NPAD = 8 * _MBLK

_f32 = jnp.float32


def _nsc(d):
    return jnp.where(d > 0, lax.rsqrt(jnp.maximum(d, 1.0)), 0.0)


def _l1_body(emb_ref, deg_ref, w_ref, o0, o1, o2):
    x = emb_ref[...]
    for i, o in enumerate((o0, o1, o2)):
        ns = _nsc(deg_ref[:, 2 * i:2 * i + 1])
        o[...] = jnp.dot(x, w_ref[i], preferred_element_type=_f32) * ns


def _l1_tc(emb304, deg, w1s):
    return pl.pallas_call(
        _l1_body,
        grid=(_NB,),
        in_specs=[
            pl.BlockSpec((_BLK, _DP), lambda i: (i, 0)),
            pl.BlockSpec((_BLK, 6), lambda i: (i, 0)),
            pl.BlockSpec((3, _DP, _DP), lambda i: (0, 0, 0)),
        ],
        out_specs=[pl.BlockSpec((_BLK, _DP), lambda i: (i, 0))] * 3,
        out_shape=[jax.ShapeDtypeStruct((_N, _DP), _f32)] * 3,
    )(emb304, deg, w1s)


def _l2_body(a0, a1, a2, deg_ref, b_ref, w_ref, o0, o1, o2):
    for i, (a, o) in enumerate(zip((a0, a1, a2), (o0, o1, o2))):
        nd = _nsc(deg_ref[:, 2 * i + 1:2 * i + 2])
        ns = _nsc(deg_ref[:, 2 * i:2 * i + 1])
        h = a[...] * nd + b_ref[i]
        o[...] = jnp.dot(h, w_ref[i], preferred_element_type=_f32) * ns


def _l2_tc(aggs, deg, b1s, w2s):
    return pl.pallas_call(
        _l2_body,
        grid=(_NB,),
        in_specs=[pl.BlockSpec((_BLK, _DP), lambda i: (i, 0))] * 3 + [
            pl.BlockSpec((_BLK, 6), lambda i: (i, 0)),
            pl.BlockSpec((3, 1, _DP), lambda i: (0, 0, 0)),
            pl.BlockSpec((3, _DP, _DP), lambda i: (0, 0, 0)),
        ],
        out_specs=[pl.BlockSpec((_BLK, _DP), lambda i: (i, 0))] * 3,
        out_shape=[jax.ShapeDtypeStruct((_N, _DP), _f32)] * 3,
    )(*aggs, deg, b1s, w2s)


def _mha_body(emb_ref, a0, a1, a2, deg_ref, b2_ref, wq_ref, wk_ref, wv_ref,
              m_ref, mt_ref, o_ref):
    iblk = pl.program_id(0)
    rows = lax.broadcasted_iota(jnp.int32, (_MBLK, 1), 0) + iblk * _MBLK
    cats = []
    x = emb_ref[...]
    for i, a in enumerate((a0, a1, a2)):
        nd = _nsc(deg_ref[:, 2 * i + 1:2 * i + 2])
        cats.append(x + a[...] * nd + b2_ref[i])
    q = [jnp.dot(c, wq_ref[...], preferred_element_type=_f32) for c in cats]
    k = [jnp.dot(c, wk_ref[...], preferred_element_type=_f32) for c in cats]
    v = [jnp.dot(c, wv_ref[...], preferred_element_type=_f32) for c in cats]
    m = m_ref[...]
    mt = mt_ref[...]
    om = jnp.zeros((_MBLK, _DP), _f32)
    for i in range(3):
        s = [jnp.dot(q[i] * k[j], m, preferred_element_type=_f32)
             for j in range(3)]
        mx = jnp.maximum(jnp.maximum(s[0], s[1]), s[2])
        e = [jnp.exp(sj - mx) for sj in s]
        den = e[0] + e[1] + e[2]
        for j in range(3):
            om = om + jnp.dot(e[j] / den, mt,
                              preferred_element_type=_f32) * v[j]
    om = om * (1.0 / 3.0)
    o_ref[...] = jnp.where(rows < _N, om, 0.0)


def _mha_tc(emb304, aggs, deg, b2s, wq, wk, wv, m, mt):
    return pl.pallas_call(
        _mha_body,
        grid=(_NPAD // _MBLK,),
        in_specs=[pl.BlockSpec((_MBLK, _DP), lambda i: (i, 0))] * 4 + [
            pl.BlockSpec((_MBLK, 6), lambda i: (i, 0)),
            pl.BlockSpec((3, 1, _DP), lambda i: (0, 0, 0)),
            pl.BlockSpec((_DP, _DP), lambda i: (0, 0)),
            pl.BlockSpec((_DP, _DP), lambda i: (0, 0)),
            pl.BlockSpec((_DP, _DP), lambda i: (0, 0)),
            pl.BlockSpec((_DP, 32), lambda i: (0, 0)),
            pl.BlockSpec((32, _DP), lambda i: (0, 0)),
        ],
        out_specs=pl.BlockSpec((_MBLK, _DP), lambda i: (i, 0)),
        out_shape=jax.ShapeDtypeStruct((_NPAD, _DP), _f32),
    )(emb304, *aggs, deg, b2s, wq, wk, wv, m, mt)


def _fin_body(ds_ref, wo_ref, wd_ref, wf_ref, bd_ref, bf_ref, o_ref):
    wdf = jnp.dot(wd_ref[...], wf_ref[...], preferred_element_type=_f32)
    wc = jnp.dot(wo_ref[...], wdf, preferred_element_type=_f32)
    bc = jnp.dot(bd_ref[...], wf_ref[...], preferred_element_type=_f32) \
        + bf_ref[...]
    o_ref[...] = jnp.dot(ds_ref[...] * (1.0 / _L), wc,
                         preferred_element_type=_f32) + bc


def _fin_tc(docsum, wo, wd, wf, bd, bf):
    return pl.pallas_call(
        _fin_body,
        out_shape=jax.ShapeDtypeStruct((_B, _C), _f32),
    )(docsum, wo, wd, wf, bd, bf)


def _red_body(p_ref, o_ref):
    o_ref[...] = jnp.sum(p_ref[...], axis=0)


def _red_tc(partials):
    return pl.pallas_call(
        _red_body,
        out_shape=jax.ShapeDtypeStruct((_N * 6,), _f32),
    )(partials)


def kernel(emb, w_dis1, b_dis1, w_dis2, b_dis2, w_pmi1, b_pmi1, w_pmi2,
           b_pmi2, w_top1, b_top1, w_top2, b_top2, wq, wk, wv, wo, wd, bd,
           wf, bf, ei_dis, ei_pmi, ei_top, adj):
    padw = lambda w: jnp.pad(w, ((0, 4), (0, 4)))
    padb = lambda b: jnp.pad(b, (0, 4)).reshape(1, _DP)
    emb304 = jnp.pad(emb, ((0, 0), (0, 4)))
    w1s = jnp.stack([padw(w_dis1), padw(w_pmi1), padw(w_top1)])
    w2s = jnp.stack([padw(w_dis2), padw(w_pmi2), padw(w_top2)])
    b1s = jnp.stack([padb(b_dis1), padb(b_pmi1), padb(b_top1)])
    b2s = jnp.stack([padb(b_dis2), padb(b_pmi2), padb(b_top2)])
    wqp, wkp, wvp = padw(wq), padw(wk), padw(wv)
    wop, wdp = padw(wo), padw(wd)
    wfp = jnp.pad(wf, ((0, 4), (0, 0)))
    bdp = jnp.pad(bd, (0, 4)).reshape(1, _DP)
    bfp = bf.reshape(1, _C)

    mnp = np.zeros((_DP, 32), np.float32)
    mnp[np.arange(_D), np.arange(_D) // (_D // _H)] = 1.0
    m = jnp.asarray(mnp / np.sqrt(_D // _H))
    mt = jnp.asarray(mnp.T.copy())

    cols = []
    for ei in (ei_dis, ei_pmi, ei_top):
        for r in range(2):
            cols.append(jnp.zeros((_N,), _f32).at[ei[r]].add(1.0))
    deg = jnp.stack(cols, axis=1)

    x1 = _l1_tc(emb304, deg, w1s)

    def spmm(xp, ei):
        return jnp.zeros((_N, _DP), _f32).at[ei[1]].add(xp[ei[0]])

    agg1 = [spmm(x1[i], ei) for i, ei in enumerate((ei_dis, ei_pmi, ei_top))]

    x2 = _l2_tc(agg1, deg, b1s, w2s)

    agg2 = [spmm(x2[i], ei) for i, ei in enumerate((ei_dis, ei_pmi, ei_top))]

    table = _mha_tc(emb304, agg2, deg, b2s, wqp, wkp, wvp, m, mt)

    docsum = table[adj].sum(axis=1)

    return _fin_tc(docsum, wop, wdp, wfp, bdp, bfp)


# speedup vs baseline: 3.6058x; 1.9370x over previous
import numpy as np

import jax
import jax.numpy as jnp
from jax import lax
from jax.experimental import pallas as pl
from jax.experimental.pallas import tpu as pltpu
from jax.experimental.pallas import tpu_sc as plsc

_N, _E, _D, _C, _B, _L, _H = 10000, 160000, 300, 20, 4096, 50, 30
_DP = 384
_W = 128
_NPART = _DP // _W
_BLK = 1000
_NB = _N // _BLK
_MBLK = 632
_NPAD = 16 * _MBLK
_SROWS = 16 * 632

_f32 = jnp.float32


def _nsc(d):
    return jnp.where(d > 0, lax.rsqrt(jnp.maximum(d, 1.0)), 0.0)


def _parts(r):
    return [r[:, p * _W:(p + 1) * _W] for p in range(_NPART)]


def _comb(prefs):
    return jnp.concatenate([p[0] + p[1] for p in prefs], axis=1)


def _l1_body(emb_ref, deg_ref, w_ref, *outs):
    x = emb_ref[...]
    for i in range(3):
        ns = _nsc(deg_ref[:, 2 * i:2 * i + 1])
        r = jnp.dot(x, w_ref[i], preferred_element_type=_f32) * ns
        for p, rp in enumerate(_parts(r)):
            outs[i * _NPART + p][...] = rp


def _l1_tc(emb384, deg, w1s):
    return pl.pallas_call(
        _l1_body,
        grid=(_NB,),
        in_specs=[
            pl.BlockSpec((_BLK, _DP), lambda i: (i, 0)),
            pl.BlockSpec((_BLK, 128), lambda i: (i, 0)),
            pl.BlockSpec((3, _DP, _DP), lambda i: (0, 0, 0)),
        ],
        out_specs=[pl.BlockSpec((_BLK, _W), lambda i: (i, 0))] * 9,
        out_shape=[jax.ShapeDtypeStruct((_N, _W), _f32)] * 9,
    )(emb384, deg, w1s)


def _l2_body(*refs):
    aggs, (deg_ref, b_ref, w_ref), outs = refs[:9], refs[9:12], refs[12:]
    for i in range(3):
        nd = _nsc(deg_ref[:, 2 * i + 1:2 * i + 2])
        ns = _nsc(deg_ref[:, 2 * i:2 * i + 1])
        a = _comb(aggs[i * _NPART:(i + 1) * _NPART])
        h = a * nd + b_ref[i]
        r = jnp.dot(h, w_ref[i], preferred_element_type=_f32) * ns
        for p, rp in enumerate(_parts(r)):
            outs[i * _NPART + p][...] = rp


def _l2_tc(aggs, deg, b1s, w2s):
    return pl.pallas_call(
        _l2_body,
        grid=(_NB,),
        in_specs=[pl.BlockSpec((2, _BLK, _W), lambda i: (0, i, 0))] * 9 + [
            pl.BlockSpec((_BLK, 128), lambda i: (i, 0)),
            pl.BlockSpec((3, 1, _DP), lambda i: (0, 0, 0)),
            pl.BlockSpec((3, _DP, _DP), lambda i: (0, 0, 0)),
        ],
        out_specs=[pl.BlockSpec((_BLK, _W), lambda i: (i, 0))] * 9,
        out_shape=[jax.ShapeDtypeStruct((_N, _W), _f32)] * 9,
    )(*aggs, deg, b1s, w2s)


def _mha_body(*refs):
    emb_ref = refs[0]
    aggs = refs[1:10]
    deg_ref, b2_ref, wq_ref, wk_ref, wv_ref, m_ref, mt_ref = refs[10:17]
    o_ref = refs[17]
    iblk = pl.program_id(0)
    rows = lax.broadcasted_iota(jnp.int32, (_MBLK, 1), 0) + iblk * _MBLK
    cats = []
    x = emb_ref[...]
    for i in range(3):
        nd = _nsc(deg_ref[:, 2 * i + 1:2 * i + 2])
        a = _comb(aggs[i * _NPART:(i + 1) * _NPART])
        cats.append(x + a * nd + b2_ref[i])
    q = [jnp.dot(c, wq_ref[...], preferred_element_type=_f32) for c in cats]
    k = [jnp.dot(c, wk_ref[...], preferred_element_type=_f32) for c in cats]
    v = [jnp.dot(c, wv_ref[...], preferred_element_type=_f32) for c in cats]
    m = m_ref[...]
    mt = mt_ref[...]
    om = jnp.zeros((_MBLK, _DP), _f32)
    for i in range(3):
        s = [jnp.dot(q[i] * k[j], m, preferred_element_type=_f32)
             for j in range(3)]
        mx = jnp.maximum(jnp.maximum(s[0], s[1]), s[2])
        e = [jnp.exp(sj - mx) for sj in s]
        den = e[0] + e[1] + e[2]
        for j in range(3):
            om = om + jnp.dot(e[j] / den, mt,
                              preferred_element_type=_f32) * v[j]
    om = om * (1.0 / 3.0)
    o_ref[...] = jnp.where(rows < _N, om, 0.0)


def _mha_tc(emb384, aggs, deg, b2s, wq, wk, wv, m, mt):
    return pl.pallas_call(
        _mha_body,
        grid=(_NPAD // _MBLK,),
        in_specs=[pl.BlockSpec((_MBLK, _DP), lambda i: (i, 0))] + [
            pl.BlockSpec((2, _MBLK, _W), lambda i: (0, i, 0))] * 9 + [
            pl.BlockSpec((_MBLK, 128), lambda i: (i, 0)),
            pl.BlockSpec((3, 1, _DP), lambda i: (0, 0, 0)),
            pl.BlockSpec((_DP, _DP), lambda i: (0, 0)),
            pl.BlockSpec((_DP, _DP), lambda i: (0, 0)),
            pl.BlockSpec((_DP, _DP), lambda i: (0, 0)),
            pl.BlockSpec((_DP, 32), lambda i: (0, 0)),
            pl.BlockSpec((32, _DP), lambda i: (0, 0)),
        ],
        out_specs=pl.BlockSpec((_MBLK, _DP), lambda i: (i, 0)),
        out_shape=jax.ShapeDtypeStruct((_NPAD, _DP), _f32),
    )(emb384, *aggs, deg, b2s, wq, wk, wv, m, mt)


def _fin_body(ds_ref, wo_ref, wd_ref, wf_ref, bd_ref, bf_ref, o_ref):
    wdf = jnp.dot(wd_ref[...], wf_ref[...], preferred_element_type=_f32)
    wc = jnp.dot(wo_ref[...], wdf, preferred_element_type=_f32)
    bc = jnp.dot(bd_ref[...], wf_ref[...], preferred_element_type=_f32) \
        + bf_ref[...]
    o_ref[...] = jnp.dot(ds_ref[...] * (1.0 / _L), wc,
                         preferred_element_type=_f32) + bc


def _fin_tc(docsum, wo, wd, wf, bd, bf):
    return pl.pallas_call(
        _fin_body,
        out_shape=jax.ShapeDtypeStruct((_B, _C), _f32),
    )(docsum, wo, wd, wf, bd, bf)


def _red_body(p_ref, o_ref):
    o_ref[...] = p_ref[0] + p_ref[1]


def _red_tc(partials):
    return pl.pallas_call(
        _red_body,
        grid=(8,),
        in_specs=[pl.BlockSpec((2, _SROWS // 8, 128), lambda i: (0, i, 0))],
        out_specs=pl.BlockSpec((_SROWS // 8, 128), lambda i: (i, 0)),
        out_shape=jax.ShapeDtypeStruct((_SROWS, 128), _f32),
    )(partials)


_HCB = 128


def _hist_body(eif_hbm, out_hbm, idx_v, oh_v, zb_v, dsh):
    c = lax.axis_index("c")
    s = lax.axis_index("s")
    zeros16 = jnp.zeros((16,), _f32)
    ones16 = jnp.ones((16,), _f32)
    lanes = lax.iota(jnp.int32, 16)

    def zrow(i, _):
        for v2 in range(_W // 16):
            zb_v[i, pl.ds(v2 * 16, 16)] = zeros16
        return 0

    lax.fori_loop(0, 16, zrow, 0)
    for rep in range(39):
        pltpu.sync_copy(zb_v, dsh.at[pl.ds(s * 632 + rep * 16, 16)])
    pltpu.sync_copy(zb_v.at[pl.ds(0, 8)], dsh.at[pl.ds(s * 632 + 624, 8)])

    def zoh(i, _):
        for v2 in range(1, _W // 16):
            oh_v[i, pl.ds(v2 * 16, 16)] = zeros16
        return 0

    lax.fori_loop(0, _HCB, zoh, 0)
    plsc.subcore_barrier()

    for aa in range(3):
        pos = c * 3 + aa
        vals = jnp.where(lanes == jnp.full((16,), pos, jnp.int32),
                         ones16, zeros16)

        def fill(i, _):
            oh_v[i, pl.ds(0, 16)] = vals
            return 0

        lax.fori_loop(0, _HCB, fill, 0)

        def chunk(i, _):
            off = pos * _E + (s + 16 * i) * _HCB
            pltpu.sync_copy(eif_hbm.at[pl.ds(off, _HCB)], idx_v)
            pltpu.sync_copy(oh_v, dsh.at[idx_v], add=True)
            return 0

        nfull = _E // _HCB // 16
        lax.fori_loop(0, nfull, chunk, 0)

        @pl.when(s < (_E // _HCB) % 16)
        def _():
            chunk(nfull, 0)

    plsc.subcore_barrier()
    pltpu.sync_copy(dsh.at[pl.ds(s * 632, 632)],
                    out_hbm.at[c, pl.ds(s * 632, 632)])


_hist_sc = pl.kernel(
    _hist_body,
    out_type=jax.ShapeDtypeStruct((2, _SROWS, _W), _f32),
    mesh=plsc.VectorSubcoreMesh(core_axis_name="c", subcore_axis_name="s"),
    scratch_types=[
        pltpu.VMEM((_HCB,), jnp.int32),
        pltpu.VMEM((_HCB, _W), _f32),
        pltpu.VMEM((16, _W), _f32),
        pltpu.VMEM_SHARED((_SROWS, _W), _f32),
    ],
)


_CB = 128
_EPC = _E // 2
_NCH = _EPC // _CB


def _spmm_body(xp_hbm, src_hbm, dst_hbm, out_hbm,
               idx_v, dst_v, rows_v, zbuf_v, ysh, sem):
    c = lax.axis_index("c")
    s = lax.axis_index("s")

    def zrow(i, _):
        for v2 in range(_W // 16):
            zbuf_v[i, pl.ds(v2 * 16, 16)] = jnp.zeros((16,), _f32)
        return 0

    lax.fori_loop(0, 16, zrow, 0)
    for rep in range(39):
        pltpu.sync_copy(zbuf_v, ysh.at[pl.ds(s * 632 + rep * 16, 16)])
    pltpu.sync_copy(zbuf_v.at[pl.ds(0, 8)], ysh.at[pl.ds(s * 632 + 624, 8)])
    plsc.subcore_barrier()

    def chunk(i, _):
        off = c * _EPC + (s + 16 * i) * _CB
        pltpu.sync_copy(src_hbm.at[pl.ds(off, _CB)], idx_v)
        pltpu.sync_copy(dst_hbm.at[pl.ds(off, _CB)], dst_v)
        pltpu.async_copy(xp_hbm.at[idx_v], rows_v, sem).wait()
        pltpu.sync_copy(rows_v, ysh.at[dst_v], add=True)
        return 0

    lax.fori_loop(0, _NCH // 16, chunk, 0)

    @pl.when(s < _NCH % 16)
    def _():
        chunk(_NCH // 16, 0)

    plsc.subcore_barrier()

    pltpu.sync_copy(ysh.at[pl.ds(s * 632, 632)],
                    out_hbm.at[c, pl.ds(s * 632, 632)])


_spmm_sc = pl.kernel(
    _spmm_body,
    out_type=jax.ShapeDtypeStruct((2, _SROWS, _W), _f32),
    mesh=plsc.VectorSubcoreMesh(core_axis_name="c", subcore_axis_name="s"),
    scratch_types=[
        pltpu.VMEM((_CB,), jnp.int32),
        pltpu.VMEM((_CB,), jnp.int32),
        pltpu.VMEM((_CB, _W), _f32),
        pltpu.VMEM((16, _W), _f32),
        pltpu.VMEM_SHARED((_SROWS, _W), _f32),
        pltpu.SemaphoreType.DMA,
    ],
)


_DPW = _B // 32
_LP = 56


def _doc_body(tab_hbm, adjf_hbm, out_hbm, adjb_v, rows_v, acc_v, sem):
    c = lax.axis_index("c")
    s = lax.axis_index("s")
    wid = s * 2 + c
    dbase = wid * _DPW
    nv = _DP // 16

    def iter8(t, _):
        doc0 = dbase + t * 8
        pltpu.sync_copy(adjf_hbm.at[pl.ds(doc0 * _LP, 8 * _LP)], adjb_v)
        for h in range(4):
            pltpu.async_copy(
                tab_hbm.at[adjb_v.at[pl.ds(h * 2 * _LP, 2 * _LP)]],
                rows_v, sem).wait()
            for dd in range(2):
                def rb(r, carry):
                    return tuple(
                        carry[v2] + rows_v[dd * _LP + r, pl.ds(v2 * 16, 16)]
                        for v2 in range(nv))

                acc = lax.fori_loop(
                    0, _LP, rb, tuple(jnp.zeros((16,), _f32)
                                      for _ in range(nv)))
                for v2 in range(nv):
                    acc_v[h * 2 + dd, pl.ds(v2 * 16, 16)] = acc[v2]
        pltpu.sync_copy(acc_v, out_hbm.at[pl.ds(doc0, 8)])
        return 0

    lax.fori_loop(0, _DPW // 8, iter8, 0)


_doc_sc = pl.kernel(
    _doc_body,
    out_type=jax.ShapeDtypeStruct((_B, _DP), _f32),
    mesh=plsc.VectorSubcoreMesh(core_axis_name="c", subcore_axis_name="s"),
    scratch_types=[
        pltpu.VMEM((8 * _LP,), jnp.int32),
        pltpu.VMEM((2 * _LP, _DP), _f32),
        pltpu.VMEM((8, _DP), _f32),
        pltpu.SemaphoreType.DMA,
    ],
)


def kernel(emb, w_dis1, b_dis1, w_dis2, b_dis2, w_pmi1, b_pmi1, w_pmi2,
           b_pmi2, w_top1, b_top1, w_top2, b_top2, wq, wk, wv, wo, wd, bd,
           wf, bf, ei_dis, ei_pmi, ei_top, adj):
    pw = _DP - _D
    padw = lambda w: jnp.pad(w, ((0, pw), (0, pw)))
    padb = lambda b: jnp.pad(b, (0, pw)).reshape(1, _DP)
    emb384 = jnp.pad(emb, ((0, 0), (0, pw)))
    w1s = jnp.stack([padw(w_dis1), padw(w_pmi1), padw(w_top1)])
    w2s = jnp.stack([padw(w_dis2), padw(w_pmi2), padw(w_top2)])
    b1s = jnp.stack([padb(b_dis1), padb(b_pmi1), padb(b_top1)])
    b2s = jnp.stack([padb(b_dis2), padb(b_pmi2), padb(b_top2)])
    wqp, wkp, wvp = padw(wq), padw(wk), padw(wv)
    wop, wdp = padw(wo), padw(wd)
    wfp = jnp.pad(wf, ((0, pw), (0, 0)))
    bdp = jnp.pad(bd, (0, pw)).reshape(1, _DP)
    bfp = bf.reshape(1, _C)

    mnp = np.zeros((_DP, 32), np.float32)
    mnp[np.arange(_D), np.arange(_D) // (_D // _H)] = 1.0
    m = jnp.asarray(mnp / np.sqrt(_D // _H))
    mt = jnp.asarray(mnp.T.copy())

    eif = jnp.concatenate([ei_dis.reshape(-1), ei_pmi.reshape(-1),
                           ei_top.reshape(-1)])
    deg = _red_tc(_hist_sc(eif))

    x1 = _l1_tc(emb384, deg, w1s)

    eis = (ei_dis, ei_pmi, ei_top)
    agg1 = [_spmm_sc(x1[i * _NPART + p], ei[0], ei[1])
            for i, ei in enumerate(eis) for p in range(_NPART)]

    x2 = _l2_tc(agg1, deg, b1s, w2s)

    agg2 = [_spmm_sc(x2[i * _NPART + p], ei[0], ei[1])
            for i, ei in enumerate(eis) for p in range(_NPART)]

    table = _mha_tc(emb384, agg2, deg, b2s, wqp, wkp, wvp, m, mt)

    adjp = jnp.pad(adj, ((0, 0), (0, _LP - _L)), constant_values=_N)
    docsum = _doc_sc(table, adjp.reshape(-1))

    return _fin_tc(docsum, wop, wdp, wfp, bdp, bfp)
